# baseline (speedup 1.0000x reference)
import jax
import jax.numpy as jnp
from jax.experimental import pallas as pl
from jax.experimental.pallas import tpu as pltpu

N = 4096
D = 128
O = 32
BR = 256
NB = N // BR
SB = 11


def _fused_kernel(adj_ref, adjs_ref, adjt_ref, inps_ref, inpt_ref, w_ref,
                  xn_ref, yn_ref,
                  stash_ref,
                  rds_ref,
                  dto_ref,
                  rdt_ref,
                  xyp_ref,
                  xs_ref,
                  yt_ref):
    i = pl.program_id(0)

    @pl.when(i < NB)
    def _phase1():
        a = adj_ref[...]
        asrc = adjs_ref[...]

        @pl.when(i < SB)
        def _():
            stash_ref[pl.ds(i * BR, BR), :] = asrc.astype(jnp.bfloat16)

        rds_ref[pl.ds(i * BR, BR), :] = (
            jnp.sum(a, axis=1, keepdims=True)
            + jnp.sum(asrc, axis=1, keepdims=True))
        csum = (jnp.sum(a, axis=0, keepdims=True)
                + jnp.sum(adjt_ref[...], axis=0, keepdims=True))

        @pl.when(i == 0)
        def _():
            dto_ref[...] = csum

        @pl.when(i > 0)
        def _():
            dto_ref[...] += csum

        xyp_ref[pl.ds(i * BR, BR), 0:O] = jnp.dot(
            inps_ref[...], w_ref[...], preferred_element_type=jnp.float32)
        xyp_ref[pl.ds(i * BR, BR), O:2 * O] = jnp.dot(
            inpt_ref[...], w_ref[...], preferred_element_type=jnp.float32)

    @pl.when(i == NB)
    def _precompute():
        rds = jnp.sqrt(rds_ref[...] + 1.0)
        rds_ref[...] = rds
        rdt = jnp.sqrt(dto_ref[...].reshape(N, 1) + 1.0)
        rdt_ref[...] = rdt
        xs_ref[...] = (xyp_ref[:, 0:O] / rds).astype(jnp.bfloat16)
        yt_ref[...] = (xyp_ref[:, O:2 * O] / rdt).astype(jnp.bfloat16)

    @pl.when(jnp.logical_and(i >= NB, i < 2 * NB))
    def _phase2():
        j = i - NB
        a = adj_ref[...].astype(jnp.bfloat16)
        s_blk = jax.lax.cond(
            j < SB,
            lambda: stash_ref[pl.ds(jnp.minimum(j, SB - 1) * BR, BR), :],
            lambda: adjs_ref[...].astype(jnp.bfloat16))
        acc = (jnp.dot(s_blk, xs_ref[...], preferred_element_type=jnp.float32)
               + jnp.dot(a, yt_ref[...], preferred_element_type=jnp.float32))
        x_blk = xyp_ref[pl.ds(j * BR, BR), 0:O]
        rds_blk = rds_ref[pl.ds(j * BR, BR), :]
        xn = jnp.maximum((x_blk + acc) / rds_blk, 0.0)
        xn_ref[...] = xn
        contrib = jax.lax.dot_general(
            a, (xn / rds_blk).astype(jnp.bfloat16),
            (((0,), (0,)), ((), ())), preferred_element_type=jnp.float32)

        @pl.when(j == 0)
        def _():
            xyp_ref[:, 2 * O:3 * O] = contrib

        @pl.when(j > 0)
        def _():
            xyp_ref[:, 2 * O:3 * O] += contrib

    @pl.when(i >= 2 * NB)
    def _phase3():
        k = i - 2 * NB
        at = adjt_ref[...].astype(jnp.bfloat16)
        acc = jnp.dot(at, yt_ref[...], preferred_element_type=jnp.float32)
        y_blk = xyp_ref[pl.ds(k * BR, BR), O:2 * O]
        yp_blk = xyp_ref[pl.ds(k * BR, BR), 2 * O:3 * O]
        rdt_blk = rdt_ref[pl.ds(k * BR, BR), :]
        yn_ref[...] = jnp.maximum((y_blk + acc + yp_blk) / rdt_blk, 0.0)


def kernel(inp_s, inp_t, adj, adj_s, adj_t, W):
    last = NB - 1
    adj_map = lambda i: (jnp.where(i < 2 * NB, i % NB, last), 0)
    adjs_map = lambda i: (
        jnp.where(i < NB, i,
                  jnp.where(jnp.logical_and(i >= NB + SB, i < 2 * NB),
                            i - NB, last)), 0)
    adjt_map = lambda i: (
        jnp.where(i < NB, i, jnp.where(i < 2 * NB, last, i - 2 * NB)), 0)
    inp_map = lambda i: (jnp.where(i < NB, i, last), 0)
    xn_map = lambda i: (
        jnp.where(i < NB, 0, jnp.where(i < 2 * NB, i - NB, last)), 0)
    yn_map = lambda i: (jnp.where(i < 2 * NB, 0, i - 2 * NB), 0)

    xn, yn = pl.pallas_call(
        _fused_kernel,
        grid=(3 * NB,),
        in_specs=[pl.BlockSpec((BR, N), adj_map),
                  pl.BlockSpec((BR, N), adjs_map),
                  pl.BlockSpec((BR, N), adjt_map),
                  pl.BlockSpec((BR, D), inp_map),
                  pl.BlockSpec((BR, D), inp_map),
                  pl.BlockSpec((D, O), lambda i: (0, 0))],
        out_specs=[pl.BlockSpec((BR, O), xn_map),
                   pl.BlockSpec((BR, O), yn_map)],
        out_shape=[jax.ShapeDtypeStruct((N, O), jnp.float32),
                   jax.ShapeDtypeStruct((N, O), jnp.float32)],
        scratch_shapes=[pltpu.VMEM((SB * BR, N), jnp.bfloat16),
                        pltpu.VMEM((N, 1), jnp.float32),
                        pltpu.VMEM((1, N), jnp.float32),
                        pltpu.VMEM((N, 1), jnp.float32),
                        pltpu.VMEM((N, 4 * O), jnp.float32),
                        pltpu.VMEM((N, O), jnp.bfloat16),
                        pltpu.VMEM((N, O), jnp.bfloat16)],
        compiler_params=pltpu.CompilerParams(
            vmem_limit_bytes=128 * 1024 * 1024),
    )(adj, adj_s, adj_t, inp_s, inp_t, W)

    return (xn, yn)
